# Initial kernel scaffold; baseline (speedup 1.0000x reference)
import functools

import jax
import jax.numpy as jnp
from jax import lax
from jax.experimental import pallas as pl
from jax.experimental.pallas import tpu as pltpu
from jax.experimental.pallas import tpu_sc as plsc

NC, NS = 2, 16
NW = NC * NS
CH = 80


def _sc_aggregate(src, dst, x, zeros):
    n, d = x.shape
    e = src.shape[0]
    epw = e // NW
    nchunk = epw // CH
    rpt = n // NS

    mesh = plsc.VectorSubcoreMesh(core_axis_name="c", subcore_axis_name="s")

    @functools.partial(
        pl.kernel,
        mesh=mesh,
        out_type=jax.ShapeDtypeStruct((NC, n, d), jnp.float32),
        scratch_types=[
            pltpu.VMEM((CH,), jnp.int32),
            pltpu.VMEM((CH,), jnp.int32),
            pltpu.VMEM((CH, d), jnp.float32),
            pltpu.VMEM_SHARED((n, d), jnp.float32),
            pltpu.SemaphoreType.DMA,
        ],
    )
    def agg_kernel(src_hbm, dst_hbm, x_hbm, zeros_hbm, out_hbm,
                   src_v, dst_v, rows_v, agg_sh, sem):
        c = lax.axis_index("c")
        s = lax.axis_index("s")
        wid = c * NS + s

        pltpu.sync_copy(zeros_hbm, agg_sh.at[pl.ds(s * rpt, rpt)])
        plsc.subcore_barrier()

        def body(i, carry):
            base = wid * epw + i * CH
            pltpu.sync_copy(src_hbm.at[pl.ds(base, CH)], src_v)
            pltpu.sync_copy(dst_hbm.at[pl.ds(base, CH)], dst_v)
            pltpu.async_copy(x_hbm.at[src_v], rows_v, sem).wait()
            pltpu.sync_copy(rows_v, agg_sh.at[dst_v], add=True)
            return carry

        lax.fori_loop(0, nchunk, body, 0)
        plsc.subcore_barrier()

        pltpu.sync_copy(agg_sh.at[pl.ds(s * rpt, rpt)],
                        out_hbm.at[c, pl.ds(s * rpt, rpt)])

    return agg_kernel(src, dst, x, zeros)


def _tc_block(x_ref, a0_ref, a1_ref, w1_ref, b1_ref, w2_ref, b2_ref,
              g_ref, be_ref, o_ref):
    xb = x_ref[...]
    h = xb + a0_ref[...] + a1_ref[...]
    t = jnp.dot(h, w1_ref[...], preferred_element_type=jnp.float32) + b1_ref[...]
    t = jnp.maximum(t, 0.0)
    t = jnp.dot(t, w2_ref[...], preferred_element_type=jnp.float32) + b2_ref[...]
    mean = jnp.mean(t, axis=-1, keepdims=True)
    cent = t - mean
    var = jnp.mean(cent * cent, axis=-1, keepdims=True)
    t = cent * lax.rsqrt(var + 1e-5) * g_ref[...] + be_ref[...]
    o_ref[...] = xb + jnp.maximum(t, 0.0)


def _tc_mlp(x, a0, a1, W1, b1, W2, b2, gamma, beta, block_rows=400):
    n, d = x.shape
    grid = (n // block_rows,)
    row_spec = pl.BlockSpec((block_rows, d), lambda i: (i, 0))
    full_spec = pl.BlockSpec((d, d), lambda i: (0, 0))
    vec_spec = pl.BlockSpec((1, d), lambda i: (0, 0))
    return pl.pallas_call(
        _tc_block,
        grid=grid,
        in_specs=[row_spec, row_spec, row_spec, full_spec, vec_spec,
                  full_spec, vec_spec, vec_spec, vec_spec],
        out_specs=row_spec,
        out_shape=jax.ShapeDtypeStruct((n, d), jnp.float32),
    )(x, a0, a1, W1, b1.reshape(1, d), W2, b2.reshape(1, d),
      gamma.reshape(1, d), beta.reshape(1, d))


def kernel(x, edge_index, W1, b1, W2, b2, gamma, beta):
    n, d = x.shape
    src = edge_index[0]
    dst = edge_index[1]
    zeros = jnp.zeros((n // NS, d), dtype=jnp.float32)
    agg = _sc_aggregate(src, dst, x, zeros)
    return _tc_mlp(x, agg[0], agg[1], W1, b1, W2, b2, gamma, beta)


# speedup vs baseline: 5.2619x; 5.2619x over previous
import functools

import jax
import jax.numpy as jnp
from jax import lax
from jax.experimental import pallas as pl
from jax.experimental.pallas import tpu as pltpu
from jax.experimental.pallas import tpu_sc as plsc

NC, NS = 2, 16
NW = NC * NS
CH = 80


def _sc_aggregate(src, dst, x, zeros):
    n, d = x.shape
    e = src.shape[0]
    epw = e // NW
    nchunk = epw // CH
    rpt = (n // NS) // 8 * 8
    tail = n - NS * rpt

    mesh = plsc.VectorSubcoreMesh(core_axis_name="c", subcore_axis_name="s")

    @functools.partial(
        pl.kernel,
        mesh=mesh,
        out_type=jax.ShapeDtypeStruct((NC, n, d), jnp.float32),
        scratch_types=[
            pltpu.VMEM((CH,), jnp.int32),
            pltpu.VMEM((CH,), jnp.int32),
            pltpu.VMEM((CH, d), jnp.float32),
            pltpu.VMEM_SHARED((n, d), jnp.float32),
            pltpu.SemaphoreType.DMA,
        ],
    )
    def agg_kernel(src_hbm, dst_hbm, x_hbm, zeros_hbm, out_hbm,
                   src_v, dst_v, rows_v, agg_sh, sem):
        c = lax.axis_index("c")
        s = lax.axis_index("s")
        wid = c * NS + s

        pltpu.sync_copy(zeros_hbm, agg_sh.at[pl.ds(s * rpt, rpt)])
        if tail:
            @pl.when(s == 0)
            def _():
                pltpu.sync_copy(zeros_hbm.at[pl.ds(0, tail)],
                                agg_sh.at[pl.ds(NS * rpt, tail)])
        plsc.subcore_barrier()

        def body(i, carry):
            base = wid * epw + i * CH
            pltpu.sync_copy(src_hbm.at[pl.ds(base, CH)], src_v)
            pltpu.sync_copy(dst_hbm.at[pl.ds(base, CH)], dst_v)
            pltpu.async_copy(x_hbm.at[src_v], rows_v, sem).wait()
            pltpu.sync_copy(rows_v, agg_sh.at[dst_v], add=True)
            return carry

        lax.fori_loop(0, nchunk, body, 0)
        plsc.subcore_barrier()

        pltpu.sync_copy(agg_sh.at[pl.ds(s * rpt, rpt)],
                        out_hbm.at[c, pl.ds(s * rpt, rpt)])
        if tail:
            @pl.when(s == 0)
            def _():
                pltpu.sync_copy(agg_sh.at[pl.ds(NS * rpt, tail)],
                                out_hbm.at[c, pl.ds(NS * rpt, tail)])

    return agg_kernel(src, dst, x, zeros)


def _tc_block(x_ref, a0_ref, a1_ref, w1_ref, b1_ref, w2_ref, b2_ref,
              g_ref, be_ref, o_ref):
    xb = x_ref[...]
    h = xb + a0_ref[...] + a1_ref[...]
    t = jnp.dot(h, w1_ref[...], preferred_element_type=jnp.float32) + b1_ref[...]
    t = jnp.maximum(t, 0.0)
    t = jnp.dot(t, w2_ref[...], preferred_element_type=jnp.float32) + b2_ref[...]
    mean = jnp.mean(t, axis=-1, keepdims=True)
    cent = t - mean
    var = jnp.mean(cent * cent, axis=-1, keepdims=True)
    t = cent * lax.rsqrt(var + 1e-5) * g_ref[...] + be_ref[...]
    o_ref[...] = xb + jnp.maximum(t, 0.0)


def _tc_mlp(x, a0, a1, W1, b1, W2, b2, gamma, beta, block_rows=400):
    n, d = x.shape
    grid = (n // block_rows,)
    row_spec = pl.BlockSpec((block_rows, d), lambda i: (i, 0))
    full_spec = pl.BlockSpec((d, d), lambda i: (0, 0))
    vec_spec = pl.BlockSpec((1, d), lambda i: (0, 0))
    return pl.pallas_call(
        _tc_block,
        grid=grid,
        in_specs=[row_spec, row_spec, row_spec, full_spec, vec_spec,
                  full_spec, vec_spec, vec_spec, vec_spec],
        out_specs=row_spec,
        out_shape=jax.ShapeDtypeStruct((n, d), jnp.float32),
    )(x, a0, a1, W1, b1.reshape(1, d), W2, b2.reshape(1, d),
      gamma.reshape(1, d), beta.reshape(1, d))


def kernel(x, edge_index, W1, b1, W2, b2, gamma, beta):
    n, d = x.shape
    src = edge_index[0]
    dst = edge_index[1]
    zeros = jnp.zeros(((n // NS) // 8 * 8, d), dtype=jnp.float32)
    agg = _sc_aggregate(src, dst, x, zeros)
    return _tc_mlp(x, agg[0], agg[1], W1, b1, W2, b2, gamma, beta)


# speedup vs baseline: 10.6320x; 2.0205x over previous
import functools

import jax
import jax.numpy as jnp
from jax import lax
from jax.experimental import pallas as pl
from jax.experimental.pallas import tpu as pltpu
from jax.experimental.pallas import tpu_sc as plsc

NC, NS = 2, 16
NW = NC * NS
CH = 80


def _sc_aggregate(src2, dst3, x, zeros):
    n, d = x.shape
    epw = src2.shape[1]
    nchunk = epw // CH
    npair = nchunk // 2
    rpt = (n // NS) // 8 * 8
    tail = n - NS * rpt

    mesh = plsc.VectorSubcoreMesh(core_axis_name="c", subcore_axis_name="s")

    @functools.partial(
        pl.kernel,
        mesh=mesh,
        out_type=jax.ShapeDtypeStruct((NC, n, d), jnp.float32),
        scratch_types=[
            pltpu.VMEM((epw,), jnp.int32),
            pltpu.VMEM((nchunk, CH), jnp.int32),
            pltpu.VMEM((CH, d), jnp.float32),
            pltpu.VMEM((CH, d), jnp.float32),
            pltpu.VMEM_SHARED((n, d), jnp.float32),
            pltpu.SemaphoreType.DMA,
            pltpu.SemaphoreType.DMA,
        ],
    )
    def agg_kernel(src_hbm, dst_hbm, x_hbm, zeros_hbm, out_hbm,
                   src_v, dst_v, rows_a, rows_b, agg_sh, sem_a, sem_b):
        c = lax.axis_index("c")
        s = lax.axis_index("s")
        wid = c * NS + s

        pltpu.sync_copy(src_hbm.at[wid], src_v)
        pltpu.sync_copy(dst_hbm.at[wid], dst_v)

        pltpu.sync_copy(zeros_hbm, agg_sh.at[pl.ds(s * rpt, rpt)])
        if tail:
            @pl.when(s == 0)
            def _():
                pltpu.sync_copy(zeros_hbm.at[pl.ds(0, tail)],
                                agg_sh.at[pl.ds(NS * rpt, tail)])
        plsc.subcore_barrier()

        def gather(i, buf, sem):
            return pltpu.async_copy(
                x_hbm.at[src_v.at[pl.ds(i * CH, CH)]], buf, sem)

        def scatter(i, buf):
            pltpu.sync_copy(buf, agg_sh.at[dst_v.at[i]], add=True)

        gather(0, rows_a, sem_a)

        def body(k, carry):
            i0 = 2 * k
            gather(i0 + 1, rows_b, sem_b)
            pltpu.make_async_copy(
                x_hbm.at[src_v.at[pl.ds(0, CH)]], rows_a, sem_a).wait()
            scatter(i0, rows_a)
            gather(i0 + 2, rows_a, sem_a)
            pltpu.make_async_copy(
                x_hbm.at[src_v.at[pl.ds(0, CH)]], rows_b, sem_b).wait()
            scatter(i0 + 1, rows_b)
            return carry

        lax.fori_loop(0, npair, body, 0)
        pltpu.make_async_copy(
            x_hbm.at[src_v.at[pl.ds(0, CH)]], rows_a, sem_a).wait()
        if nchunk % 2:
            scatter(nchunk - 1, rows_a)
        else:
            pass
        plsc.subcore_barrier()

        pltpu.sync_copy(agg_sh.at[pl.ds(s * rpt, rpt)],
                        out_hbm.at[c, pl.ds(s * rpt, rpt)])
        if tail:
            @pl.when(s == 0)
            def _():
                pltpu.sync_copy(agg_sh.at[pl.ds(NS * rpt, tail)],
                                out_hbm.at[c, pl.ds(NS * rpt, tail)])

    return agg_kernel(src2, dst3, x, zeros)


def _tc_block(x_ref, a0_ref, a1_ref, w1_ref, b1_ref, w2_ref, b2_ref,
              g_ref, be_ref, o_ref):
    xb = x_ref[...]
    h = xb + a0_ref[...] + a1_ref[...]
    t = jnp.dot(h, w1_ref[...], preferred_element_type=jnp.float32) + b1_ref[...]
    t = jnp.maximum(t, 0.0)
    t = jnp.dot(t, w2_ref[...], preferred_element_type=jnp.float32) + b2_ref[...]
    mean = jnp.mean(t, axis=-1, keepdims=True)
    cent = t - mean
    var = jnp.mean(cent * cent, axis=-1, keepdims=True)
    t = cent * lax.rsqrt(var + 1e-5) * g_ref[...] + be_ref[...]
    o_ref[...] = xb + jnp.maximum(t, 0.0)


def _tc_mlp(x, a0, a1, W1, b1, W2, b2, gamma, beta, block_rows=400):
    n, d = x.shape
    grid = (n // block_rows,)
    row_spec = pl.BlockSpec((block_rows, d), lambda i: (i, 0))
    full_spec = pl.BlockSpec((d, d), lambda i: (0, 0))
    vec_spec = pl.BlockSpec((1, d), lambda i: (0, 0))
    return pl.pallas_call(
        _tc_block,
        grid=grid,
        in_specs=[row_spec, row_spec, row_spec, full_spec, vec_spec,
                  full_spec, vec_spec, vec_spec, vec_spec],
        out_specs=row_spec,
        out_shape=jax.ShapeDtypeStruct((n, d), jnp.float32),
    )(x, a0, a1, W1, b1.reshape(1, d), W2, b2.reshape(1, d),
      gamma.reshape(1, d), beta.reshape(1, d))


def kernel(x, edge_index, W1, b1, W2, b2, gamma, beta):
    n, d = x.shape
    e = edge_index.shape[1]
    epw = e // NW
    nchunk = epw // CH
    assert nchunk % 2 == 1 and nchunk * CH == epw and epw * NW == e
    src2 = edge_index[0].reshape(NW, epw)
    dst3 = edge_index[1].reshape(NW, nchunk, CH)
    zeros = jnp.zeros(((n // NS) // 8 * 8, d), dtype=jnp.float32)
    agg = _sc_aggregate(src2, dst3, x, zeros)
    return _tc_mlp(x, agg[0], agg[1], W1, b1, W2, b2, gamma, beta)
